# baseline (speedup 1.0000x reference)
import functools

import jax
import jax.numpy as jnp
from jax import lax
from jax.experimental import pallas as pl
from jax.experimental.pallas import tpu as pltpu
from jax.experimental.pallas import tpu_sc as plsc

G, H, W = 10, 30, 30
D = 1024
D4 = D // 4
L = 16
NC, NS = 2, 16
H_PER_W = 10
STRIP = W * D


def _sc_body(row_hbm, col_hbm, io_hbm, pair_hbm, out_hbm,
             buf, row_v, col_v, io_v, pair_v):
    wid = lax.axis_index("s") * NC + lax.axis_index("c")

    @pl.when(wid < G * (H // H_PER_W))
    def _():
        g = wid // (H // H_PER_W)
        h0 = (wid % (H // H_PER_W)) * H_PER_W

        pltpu.sync_copy(row_hbm.at[pl.ds(h0 * D4, H_PER_W * D4)], row_v)
        pltpu.sync_copy(col_hbm.at[pl.ds(0, W * D4)], col_v)
        pltpu.sync_copy(io_hbm.at[pl.ds((g % 2) * D4, D4)], io_v)
        pltpu.sync_copy(pair_hbm.at[pl.ds((g // 2) * D4, D4)], pair_v)

        def w_body(w, carry):
            base = w * D
            for c in range(D4 // L):
                off = c * L
                buf[pl.ds(base + D4 + off, L)] = col_v[pl.ds(w * D4 + off, L)]
                buf[pl.ds(base + 2 * D4 + off, L)] = io_v[pl.ds(off, L)]
                buf[pl.ds(base + 3 * D4 + off, L)] = pair_v[pl.ds(off, L)]
            return carry
        lax.fori_loop(0, W, w_body, 0)

        def h_body(i, carry):
            def c_body(c, inner):
                v = row_v[pl.ds(i * D4 + c * L, L)]
                def wb(w, acc):
                    buf[pl.ds(w * D + c * L, L)] = v
                    return acc
                return lax.fori_loop(0, W, wb, inner)
            lax.fori_loop(0, D4 // L, c_body, 0)
            pltpu.sync_copy(
                buf, out_hbm.at[pl.ds((g * H + h0 + i) * STRIP, STRIP)])
            return carry
        lax.fori_loop(0, H_PER_W, h_body, 0)


_sc_call = functools.partial(
    pl.kernel,
    out_type=jax.ShapeDtypeStruct((G * H * STRIP,), jnp.float32),
    mesh=plsc.VectorSubcoreMesh(core_axis_name="c", subcore_axis_name="s"),
    scratch_types=[
        pltpu.VMEM((STRIP,), jnp.float32),
        pltpu.VMEM((H_PER_W * D4,), jnp.float32),
        pltpu.VMEM((W * D4,), jnp.float32),
        pltpu.VMEM((D4,), jnp.float32),
        pltpu.VMEM((D4,), jnp.float32),
    ],
)


def kernel(x, row_table, col_table, io_table, pair_table):
    flat = _sc_call(_sc_body)(
        row_table.reshape(-1), col_table.reshape(-1),
        io_table.reshape(-1), pair_table.reshape(-1))
    return flat.reshape(G, H, W, D).astype(x.dtype)
